# baseline (speedup 1.0000x reference)
import functools

import jax
import jax.numpy as jnp
import numpy as np
from jax import lax
from jax.experimental import pallas as pl
from jax.experimental.pallas import tpu as pltpu
from jax.experimental.pallas import tpu_sc as plsc

B, N, H, W = 2, 4, 96, 96
P = H * W
S = 4096
THRESHOLD = 0.3
NPAIR = 6
PB = NPAIR * B
NTASK = PB * 2
NWORK = 32
ROWS_PER_WORKER = S // NWORK
CHUNK = 16
BIG = 2**30


def _task_maps():
    tags, bvs = [], []
    for i in range(N):
        for j in range(i + 1, N):
            for b in range(B):
                tag = ((i * N + j) * B + b) * 2
                for side, (v, t) in enumerate(((i, tag), (j, tag + 1))):
                    tags.append(t)
                    bvs.append(b * N + v)
    return tags, bvs


def _const_perms():
    tags, _ = _task_maps()
    tags_a = jnp.array(tags, dtype=jnp.uint32)
    skey = jax.random.key(42)
    keys_t = jax.vmap(lambda t: jax.random.fold_in(skey, t))(tags_a)
    s1 = jax.vmap(jax.random.split)(keys_t)
    s2 = jax.vmap(jax.random.split)(s1[:, 0])
    k1 = jax.vmap(lambda k: jax.random.bits(k, (P,), jnp.uint32))(s1[:, 1])
    k2 = jax.vmap(lambda k: jax.random.bits(k, (P,), jnp.uint32))(s2[:, 1])
    kk = jnp.concatenate([k1, k2], axis=0)
    payload = jnp.broadcast_to(jnp.arange(P, dtype=jnp.int32), (2 * NTASK, P))
    _, a = lax.sort_key_val(kk, payload, 1)
    perm = jnp.take_along_axis(a[:NTASK], a[NTASK:, :S], axis=1)
    return np.asarray(perm)


_PERMS = _const_perms()


def _setup(masks, pointmaps):
    _, bvs = _task_maps()
    bvs_a = jnp.array(bvs, dtype=jnp.int32)

    masks_flat = masks.reshape(B * N, P)
    pts_flat = pointmaps.reshape(B * N, P * 3)
    p3 = pointmaps.reshape(B * N, P, 3)
    valid8 = (jnp.sum(p3 * p3, axis=-1) > 1e-12)
    n8 = jnp.sum(valid8.astype(jnp.int32), axis=1)
    cs = jnp.cumsum(valid8.astype(jnp.int32), axis=1)
    pos = jnp.where(valid8, cs - 1, P)
    cols = jnp.broadcast_to(jnp.arange(P, dtype=jnp.int32), (B * N, P))
    idx_sorted8 = jnp.zeros((B * N, P), jnp.int32).at[
        jnp.arange(B * N)[:, None], pos].set(cols, mode="drop")

    perm = jnp.asarray(_PERMS)
    idx_sorted_t = idx_sorted8[bvs_a]
    n_t = n8[bvs_a]
    take = (n_t > S)[:, None]
    idx_all = jnp.where(take,
                        jnp.take_along_axis(idx_sorted_t, perm, axis=1),
                        idx_sorted_t[:, :S]).astype(jnp.int32)
    w_all = jnp.where(take, 1.0,
                      (jnp.arange(S)[None, :] < n_t[:, None]).astype(jnp.float32))

    return (masks_flat, pts_flat), idx_all, w_all, n_t


def _sc_gather(tbls, idx_all, w_all):
    info = plsc.get_sparse_core_info()
    nc = info.num_cores
    mesh = plsc.VectorSubcoreMesh(core_axis_name="c", subcore_axis_name="s")

    @functools.partial(
        pl.kernel,
        out_type=[
            jax.ShapeDtypeStruct((PB, S * 4), jnp.float32),
            jax.ShapeDtypeStruct((PB, 4, S), jnp.float32),
            jax.ShapeDtypeStruct((PB, S), jnp.float32),
            jax.ShapeDtypeStruct((PB, S), jnp.float32),
        ],
        mesh=mesh,
        compiler_params=pltpu.CompilerParams(needs_layout_passes=False),
        scratch_types=[
            pltpu.VMEM((P,), jnp.float32),
            pltpu.VMEM((P * 3,), jnp.float32),
            pltpu.VMEM((S,), jnp.int32),
            pltpu.VMEM((S,), jnp.float32),
            pltpu.VMEM((S * 4,), jnp.float32),
            pltpu.VMEM((S,), jnp.float32),
            pltpu.VMEM((S,), jnp.float32),
            pltpu.VMEM((S,), jnp.float32),
            pltpu.VMEM((S,), jnp.float32),
        ],
    )
    def k(mt_hbm, p3_hbm, idx_hbm, w_hbm,
          pp_hbm, pt_hbm, mi_hbm, mj_hbm,
          mt_v, p3_v, idx_v, w_v, pk_v, m_v, x_v, y_v, z_v):
        wid = lax.axis_index("s") * nc + lax.axis_index("c")

        @pl.when(wid < NTASK)
        def _():
            pb = wid // 2
            pair = wid // 4
            b = (wid // 2) % 2
            side = wid % 2
            vi = jnp.where(pair < 3, 0, jnp.where(pair < 5, 1, 2))
            vj = jnp.where(pair < 3, pair + 1, jnp.where(pair == 3, 2, 3))
            bv = b * N + jnp.where(side == 0, vi, vj)
            pltpu.sync_copy(mt_hbm.at[bv], mt_v)
            pltpu.sync_copy(p3_hbm.at[bv], p3_v)
            pltpu.sync_copy(idx_hbm.at[wid], idx_v)
            pltpu.sync_copy(w_hbm.at[wid], w_v)

            def chunk(c, carry):
                sl = pl.ds(c * CHUNK, CHUNK)
                ids = idx_v[sl]
                ids3 = ids * 3
                m_v[sl] = plsc.load_gather(mt_v, [ids])
                x_v[sl] = plsc.load_gather(p3_v, [ids3])
                y_v[sl] = plsc.load_gather(p3_v, [ids3 + 1])
                z_v[sl] = plsc.load_gather(p3_v, [ids3 + 2])
                return carry

            lax.fori_loop(0, S // CHUNK, chunk, 0)

            @pl.when(wid % 2 == 0)
            def _():
                def chunk2(c, carry):
                    sl = pl.ds(c * CHUNK, CHUNK)
                    pos = (lax.iota(jnp.int32, CHUNK) + c * CHUNK) * 4
                    plsc.store_scatter(pk_v, [pos], x_v[sl])
                    plsc.store_scatter(pk_v, [pos + 1], y_v[sl])
                    plsc.store_scatter(pk_v, [pos + 2], z_v[sl])
                    plsc.store_scatter(pk_v, [pos + 3], w_v[sl])
                    return carry

                lax.fori_loop(0, S // CHUNK, chunk2, 0)
                pltpu.sync_copy(pk_v, pp_hbm.at[pb])
                pltpu.sync_copy(m_v, mi_hbm.at[pb])

            @pl.when(wid % 2 == 1)
            def _():
                pltpu.sync_copy(x_v, pt_hbm.at[pb, 0])
                pltpu.sync_copy(y_v, pt_hbm.at[pb, 1])
                pltpu.sync_copy(z_v, pt_hbm.at[pb, 2])
                pltpu.sync_copy(w_v, pt_hbm.at[pb, 3])
                pltpu.sync_copy(m_v, mj_hbm.at[pb])

    return k(tbls[0], tbls[1], idx_all, w_all)


_JT, _IT = 2, 2
_JB, _IB = S // _JT, S // _IT


def _tc_body(pp_ref, pt_ref, rmin_ref, ridx_ref, nnb_ref):
    rmin_ref[...] = jnp.full((1, S, 1), jnp.inf, jnp.float32)
    ridx_ref[...] = jnp.zeros((1, S, 1), jnp.int32)

    def jt_body(jt, carry0):
        jsl = pl.ds(jt * _JB, _JB)
        xj = pt_ref[0, 0:1, jsl]
        yj = pt_ref[0, 1:2, jsl]
        zj = pt_ref[0, 2:3, jsl]
        wj = pt_ref[0, 3:4, jsl] > 0.0

        def it_body(it, carry):
            cmin, cidx = carry
            isl = pl.ds(it * _IB, _IB)
            xi = pp_ref[0, isl, 0:1]
            yi = pp_ref[0, isl, 1:2]
            zi = pp_ref[0, isl, 2:3]
            wi = pp_ref[0, isl, 3:4] > 0.0
            dx = xi - xj
            dy = yi - yj
            dz = zi - zj
            d2 = dx * dx + dy * dy + dz * dz
            dj = jnp.where(wj, d2, jnp.inf)
            tmin = jnp.min(dj, axis=1, keepdims=True)
            lane = lax.broadcasted_iota(jnp.int32, (_IB, _JB), 1)
            tidx = jnp.min(jnp.where(dj == tmin, lane, BIG),
                           axis=1, keepdims=True) + jt * _JB
            prev = rmin_ref[0, isl, :]
            pidx = ridx_ref[0, isl, :]
            better = tmin < prev
            rmin_ref[0, isl, :] = jnp.where(better, tmin, prev)
            ridx_ref[0, isl, :] = jnp.where(better, tidx, pidx)
            di = jnp.where(wi, d2, jnp.inf)
            ccmin = jnp.min(di, axis=0, keepdims=True)
            row = lax.broadcasted_iota(jnp.int32, (_IB, _JB), 0)
            ccidx = jnp.min(jnp.where(di == ccmin, row, BIG),
                            axis=0, keepdims=True) + it * _IB
            cbetter = ccmin < cmin
            return (jnp.where(cbetter, ccmin, cmin),
                    jnp.where(cbetter, ccidx, cidx))

        cmin0 = jnp.full((1, _JB), jnp.inf, jnp.float32)
        cidx0 = jnp.zeros((1, _JB), jnp.int32)
        _, cidx = lax.fori_loop(0, _IT, it_body, (cmin0, cidx0))
        nnb_ref[0, 0:1, jsl] = cidx
        return carry0

    lax.fori_loop(0, _JT, jt_body, 0)
    rmin_ref[...] = jnp.sqrt(rmin_ref[...] + 1e-12)


def _tc_nn(pts_p, pts_t):
    return pl.pallas_call(
        _tc_body,
        grid=(PB,),
        in_specs=[
            pl.BlockSpec((1, S, 4), lambda p: (p, 0, 0)),
            pl.BlockSpec((1, 4, S), lambda p: (p, 0, 0)),
        ],
        out_specs=[
            pl.BlockSpec((1, S, 1), lambda p: (p, 0, 0)),
            pl.BlockSpec((1, S, 1), lambda p: (p, 0, 0)),
            pl.BlockSpec((1, 1, S), lambda p: (p, 0, 0)),
        ],
        out_shape=[
            jax.ShapeDtypeStruct((PB, S, 1), jnp.float32),
            jax.ShapeDtypeStruct((PB, S, 1), jnp.int32),
            jax.ShapeDtypeStruct((PB, 1, S), jnp.int32),
        ],
    )(pts_p, pts_t)


def _sc_loss(nn_idx, nn_back, rmin, w_all, mask_i, mask_j):
    info = plsc.get_sparse_core_info()
    nc = info.num_cores
    mesh = plsc.VectorSubcoreMesh(core_axis_name="c", subcore_axis_name="s")

    @functools.partial(
        pl.kernel,
        out_type=[
            jax.ShapeDtypeStruct((NWORK, PB, CHUNK), jnp.float32),
            jax.ShapeDtypeStruct((NWORK, PB, CHUNK), jnp.float32),
        ],
        mesh=mesh,
        compiler_params=pltpu.CompilerParams(needs_layout_passes=False),
        scratch_types=[
            pltpu.VMEM((S,), jnp.int32),
            pltpu.VMEM((S,), jnp.float32),
            pltpu.VMEM((ROWS_PER_WORKER,), jnp.int32),
            pltpu.VMEM((ROWS_PER_WORKER,), jnp.float32),
            pltpu.VMEM((ROWS_PER_WORKER,), jnp.float32),
            pltpu.VMEM((ROWS_PER_WORKER,), jnp.float32),
            pltpu.VMEM((CHUNK,), jnp.float32),
            pltpu.VMEM((CHUNK,), jnp.float32),
        ],
    )
    def k(nni_hbm, nnb_hbm, rmin_hbm, w_hbm, mi_hbm, mj_hbm,
          outw_hbm, outl_hbm,
          nnb_v, mj_v, nni_v, rm_v, wi_v, mi_v, aw_v, al_v):
        wid = lax.axis_index("s") * nc + lax.axis_index("c")
        base = wid * ROWS_PER_WORKER
        for pb in range(PB):
            pltpu.sync_copy(nnb_hbm.at[pb], nnb_v)
            pltpu.sync_copy(mj_hbm.at[pb], mj_v)
            pltpu.sync_copy(nni_hbm.at[pb, pl.ds(base, ROWS_PER_WORKER)], nni_v)
            pltpu.sync_copy(rmin_hbm.at[pb, pl.ds(base, ROWS_PER_WORKER)], rm_v)
            pltpu.sync_copy(w_hbm.at[2 * pb, pl.ds(base, ROWS_PER_WORKER)], wi_v)
            pltpu.sync_copy(mi_hbm.at[pb, pl.ds(base, ROWS_PER_WORKER)], mi_v)

            def chunk(c, carry):
                aw, al = carry
                sl = pl.ds(c * CHUNK, CHUNK)
                ids = nni_v[sl]
                nb = plsc.load_gather(nnb_v, [ids])
                mj = plsc.load_gather(mj_v, [ids])
                r = lax.iota(jnp.int32, CHUNK) + (base + c * CHUNK)
                valid = ((nb == r)
                         & (rm_v[sl] < THRESHOLD)
                         & (wi_v[sl] > 0.0))
                wv = jnp.where(valid, 1.0, 0.0).astype(jnp.float32)
                d = mi_v[sl] - mj
                return aw + wv, al + wv * d * d

            zero = jnp.zeros((CHUNK,), jnp.float32)
            aw, al = lax.fori_loop(0, ROWS_PER_WORKER // CHUNK, chunk,
                                   (zero, zero))
            aw_v[...] = aw
            al_v[...] = al
            pltpu.sync_copy(aw_v, outw_hbm.at[wid, pb])
            pltpu.sync_copy(al_v, outl_hbm.at[wid, pb])

    return k(nn_idx, nn_back, rmin, w_all, mask_i, mask_j)


def _finish(outw, outl, n_all):
    sw = outw.sum(axis=(0, 2))
    sl = outl.sum(axis=(0, 2))
    lpb = sl / jnp.maximum(sw, 1.0)
    inc = (n_all[0::2] >= 10) & (n_all[1::2] >= 10)
    lpb = jnp.where(inc, lpb, 0.0)
    lpair = lpb.reshape(NPAIR, B).sum(axis=1)
    cnt = inc.reshape(NPAIR, B).sum(axis=1)
    tot = jnp.where(cnt > 0, lpair / jnp.maximum(cnt, 1), 0.0).sum()
    return tot / NPAIR


def kernel(masks, pointmaps):
    tbls, idx_all, w_all, n_all = _setup(masks, pointmaps)
    pts_p, pts_t, mask_i, mask_j = _sc_gather(tbls, idx_all, w_all)
    rmin, ridx, nnb = _tc_nn(pts_p.reshape(PB, S, 4), pts_t)
    outw, outl = _sc_loss(ridx.reshape(PB, S), nnb.reshape(PB, S),
                          rmin.reshape(PB, S), w_all, mask_i, mask_j)
    return _finish(outw, outl, n_all)


# speedup vs baseline: 1.0586x; 1.0586x over previous
import functools

import jax
import jax.numpy as jnp
import numpy as np
from jax import lax
from jax.experimental import pallas as pl
from jax.experimental.pallas import tpu as pltpu
from jax.experimental.pallas import tpu_sc as plsc

B, N, H, W = 2, 4, 96, 96
P = H * W
S = 4096
THRESHOLD = 0.3
NPAIR = 6
PB = NPAIR * B
NTASK = PB * 2
NWORK = 32
ROWS_PER_WORKER = S // NWORK
CHUNK = 16
BIG = 2**30


def _task_maps():
    tags, bvs = [], []
    for i in range(N):
        for j in range(i + 1, N):
            for b in range(B):
                tag = ((i * N + j) * B + b) * 2
                for side, (v, t) in enumerate(((i, tag), (j, tag + 1))):
                    tags.append(t)
                    bvs.append(b * N + v)
    return tags, bvs


def _const_perms():
    tags, _ = _task_maps()
    tags_a = jnp.array(tags, dtype=jnp.uint32)
    skey = jax.random.key(42)
    keys_t = jax.vmap(lambda t: jax.random.fold_in(skey, t))(tags_a)
    s1 = jax.vmap(jax.random.split)(keys_t)
    s2 = jax.vmap(jax.random.split)(s1[:, 0])
    k1 = jax.vmap(lambda k: jax.random.bits(k, (P,), jnp.uint32))(s1[:, 1])
    k2 = jax.vmap(lambda k: jax.random.bits(k, (P,), jnp.uint32))(s2[:, 1])
    kk = jnp.concatenate([k1, k2], axis=0)
    payload = jnp.broadcast_to(jnp.arange(P, dtype=jnp.int32), (2 * NTASK, P))
    _, a = lax.sort_key_val(kk, payload, 1)
    perm = jnp.take_along_axis(a[:NTASK], a[NTASK:, :S], axis=1)
    return np.asarray(perm)


_PERMS = _const_perms()


def _setup(masks, pointmaps):
    _, bvs = _task_maps()
    bvs_a = jnp.array(bvs, dtype=jnp.int32)

    masks_flat = masks.reshape(B * N, P)
    p3 = pointmaps.reshape(B * N, P, 3)
    valid8 = (jnp.sum(p3 * p3, axis=-1) > 1e-12)
    n8 = jnp.sum(valid8.astype(jnp.int32), axis=1)
    cs = jnp.cumsum(valid8.astype(jnp.int32), axis=1)
    pos = jnp.where(valid8, cs - 1, P)
    cols = jnp.broadcast_to(jnp.arange(P, dtype=jnp.int32), (B * N, P))
    idx_sorted8 = jnp.zeros((B * N, P), jnp.int32).at[
        jnp.arange(B * N)[:, None], pos].set(cols, mode="drop")

    perm = jnp.asarray(_PERMS)
    idx_sorted_t = idx_sorted8[bvs_a]
    n_t = n8[bvs_a]
    take = (n_t > S)[:, None]
    idx_all = jnp.where(take,
                        jnp.take_along_axis(idx_sorted_t, perm, axis=1),
                        idx_sorted_t[:, :S]).astype(jnp.int32)
    w_all = jnp.where(take, 1.0,
                      (jnp.arange(S)[None, :] < n_t[:, None]).astype(jnp.float32))

    return (masks_flat, p3[:, :, 0], p3[:, :, 1], p3[:, :, 2]), idx_all, w_all, n_t


def _sc_gather(tbls, idx_all, w_all):
    info = plsc.get_sparse_core_info()
    nc = info.num_cores
    mesh = plsc.VectorSubcoreMesh(core_axis_name="c", subcore_axis_name="s")

    @functools.partial(
        pl.kernel,
        out_type=[
            jax.ShapeDtypeStruct((PB, S * 4), jnp.float32),
            jax.ShapeDtypeStruct((PB, 4, S), jnp.float32),
            jax.ShapeDtypeStruct((PB, S), jnp.float32),
            jax.ShapeDtypeStruct((PB, S), jnp.float32),
        ],
        mesh=mesh,
        compiler_params=pltpu.CompilerParams(needs_layout_passes=False),
        scratch_types=[
            pltpu.VMEM((P,), jnp.float32),
            pltpu.VMEM((P,), jnp.float32),
            pltpu.VMEM((P,), jnp.float32),
            pltpu.VMEM((P,), jnp.float32),
            pltpu.VMEM((S,), jnp.int32),
            pltpu.VMEM((S,), jnp.float32),
            pltpu.VMEM((S * 4,), jnp.float32),
            pltpu.VMEM((S,), jnp.float32),
            pltpu.VMEM((S,), jnp.float32),
            pltpu.VMEM((S,), jnp.float32),
            pltpu.VMEM((S,), jnp.float32),
        ],
    )
    def k(mt_hbm, xt_hbm, yt_hbm, zt_hbm, idx_hbm, w_hbm,
          pp_hbm, pt_hbm, mi_hbm, mj_hbm,
          mt_v, xt_v, yt_v, zt_v, idx_v, w_v, pk_v, m_v, x_v, y_v, z_v):
        wid = lax.axis_index("s") * nc + lax.axis_index("c")

        @pl.when(wid < NTASK)
        def _():
            pb = wid // 2
            pair = wid // 4
            b = (wid // 2) % 2
            side = wid % 2
            vi = jnp.where(pair < 3, 0, jnp.where(pair < 5, 1, 2))
            vj = jnp.where(pair < 3, pair + 1, jnp.where(pair == 3, 2, 3))
            bv = b * N + jnp.where(side == 0, vi, vj)
            pltpu.sync_copy(mt_hbm.at[bv], mt_v)
            pltpu.sync_copy(xt_hbm.at[bv], xt_v)
            pltpu.sync_copy(yt_hbm.at[bv], yt_v)
            pltpu.sync_copy(zt_hbm.at[bv], zt_v)
            pltpu.sync_copy(idx_hbm.at[wid], idx_v)
            pltpu.sync_copy(w_hbm.at[wid], w_v)

            def chunk(c, carry):
                sl = pl.ds(c * CHUNK, CHUNK)
                ids = idx_v[sl]
                m_v[sl] = plsc.load_gather(mt_v, [ids])
                x_v[sl] = plsc.load_gather(xt_v, [ids])
                y_v[sl] = plsc.load_gather(yt_v, [ids])
                z_v[sl] = plsc.load_gather(zt_v, [ids])
                return carry

            lax.fori_loop(0, S // CHUNK, chunk, 0)

            @pl.when(wid % 2 == 0)
            def _():
                def chunk2(c, carry):
                    sl = pl.ds(c * CHUNK, CHUNK)
                    pos = (lax.iota(jnp.int32, CHUNK) + c * CHUNK) * 4
                    plsc.store_scatter(pk_v, [pos], x_v[sl])
                    plsc.store_scatter(pk_v, [pos + 1], y_v[sl])
                    plsc.store_scatter(pk_v, [pos + 2], z_v[sl])
                    plsc.store_scatter(pk_v, [pos + 3], w_v[sl])
                    return carry

                lax.fori_loop(0, S // CHUNK, chunk2, 0)
                pltpu.sync_copy(pk_v, pp_hbm.at[pb])
                pltpu.sync_copy(m_v, mi_hbm.at[pb])

            @pl.when(wid % 2 == 1)
            def _():
                pltpu.sync_copy(x_v, pt_hbm.at[pb, 0])
                pltpu.sync_copy(y_v, pt_hbm.at[pb, 1])
                pltpu.sync_copy(z_v, pt_hbm.at[pb, 2])
                pltpu.sync_copy(w_v, pt_hbm.at[pb, 3])
                pltpu.sync_copy(m_v, mj_hbm.at[pb])

    return k(tbls[0], tbls[1], tbls[2], tbls[3], idx_all, w_all)


_JT, _IT = 2, 2
_JB, _IB = S // _JT, S // _IT


def _tc_body(pp_ref, pt_ref, rmin_ref, ridx_ref, nnb_ref):
    rmin_ref[...] = jnp.full((1, S, 1), jnp.inf, jnp.float32)
    ridx_ref[...] = jnp.zeros((1, S, 1), jnp.int32)

    def jt_body(jt, carry0):
        jsl = pl.ds(jt * _JB, _JB)
        xj = pt_ref[0, 0:1, jsl]
        yj = pt_ref[0, 1:2, jsl]
        zj = pt_ref[0, 2:3, jsl]
        wj = pt_ref[0, 3:4, jsl] > 0.0

        def it_body(it, carry):
            cmin, cidx = carry
            isl = pl.ds(it * _IB, _IB)
            xi = pp_ref[0, isl, 0:1]
            yi = pp_ref[0, isl, 1:2]
            zi = pp_ref[0, isl, 2:3]
            wi = pp_ref[0, isl, 3:4] > 0.0
            dx = xi - xj
            dy = yi - yj
            dz = zi - zj
            d2 = dx * dx + dy * dy + dz * dz
            dj = jnp.where(wj, d2, jnp.inf)
            tmin = jnp.min(dj, axis=1, keepdims=True)
            lane = lax.broadcasted_iota(jnp.int32, (_IB, _JB), 1)
            tidx = jnp.min(jnp.where(dj == tmin, lane, BIG),
                           axis=1, keepdims=True) + jt * _JB
            prev = rmin_ref[0, isl, :]
            pidx = ridx_ref[0, isl, :]
            better = tmin < prev
            rmin_ref[0, isl, :] = jnp.where(better, tmin, prev)
            ridx_ref[0, isl, :] = jnp.where(better, tidx, pidx)
            di = jnp.where(wi, d2, jnp.inf)
            ccmin = jnp.min(di, axis=0, keepdims=True)
            row = lax.broadcasted_iota(jnp.int32, (_IB, _JB), 0)
            ccidx = jnp.min(jnp.where(di == ccmin, row, BIG),
                            axis=0, keepdims=True) + it * _IB
            cbetter = ccmin < cmin
            return (jnp.where(cbetter, ccmin, cmin),
                    jnp.where(cbetter, ccidx, cidx))

        cmin0 = jnp.full((1, _JB), jnp.inf, jnp.float32)
        cidx0 = jnp.zeros((1, _JB), jnp.int32)
        _, cidx = lax.fori_loop(0, _IT, it_body, (cmin0, cidx0))
        nnb_ref[0, 0:1, jsl] = cidx
        return carry0

    lax.fori_loop(0, _JT, jt_body, 0)
    rmin_ref[...] = jnp.sqrt(rmin_ref[...] + 1e-12)


def _tc_nn(pts_p, pts_t):
    return pl.pallas_call(
        _tc_body,
        grid=(PB,),
        in_specs=[
            pl.BlockSpec((1, S, 4), lambda p: (p, 0, 0)),
            pl.BlockSpec((1, 4, S), lambda p: (p, 0, 0)),
        ],
        out_specs=[
            pl.BlockSpec((1, S, 1), lambda p: (p, 0, 0)),
            pl.BlockSpec((1, S, 1), lambda p: (p, 0, 0)),
            pl.BlockSpec((1, 1, S), lambda p: (p, 0, 0)),
        ],
        out_shape=[
            jax.ShapeDtypeStruct((PB, S, 1), jnp.float32),
            jax.ShapeDtypeStruct((PB, S, 1), jnp.int32),
            jax.ShapeDtypeStruct((PB, 1, S), jnp.int32),
        ],
    )(pts_p, pts_t)


def _sc_loss(nn_idx, nn_back, rmin, w_all, mask_i, mask_j):
    info = plsc.get_sparse_core_info()
    nc = info.num_cores
    mesh = plsc.VectorSubcoreMesh(core_axis_name="c", subcore_axis_name="s")

    @functools.partial(
        pl.kernel,
        out_type=[
            jax.ShapeDtypeStruct((NWORK, PB, CHUNK), jnp.float32),
            jax.ShapeDtypeStruct((NWORK, PB, CHUNK), jnp.float32),
        ],
        mesh=mesh,
        compiler_params=pltpu.CompilerParams(needs_layout_passes=False),
        scratch_types=[
            pltpu.VMEM((S,), jnp.int32),
            pltpu.VMEM((S,), jnp.float32),
            pltpu.VMEM((ROWS_PER_WORKER,), jnp.int32),
            pltpu.VMEM((ROWS_PER_WORKER,), jnp.float32),
            pltpu.VMEM((ROWS_PER_WORKER,), jnp.float32),
            pltpu.VMEM((ROWS_PER_WORKER,), jnp.float32),
            pltpu.VMEM((CHUNK,), jnp.float32),
            pltpu.VMEM((CHUNK,), jnp.float32),
        ],
    )
    def k(nni_hbm, nnb_hbm, rmin_hbm, w_hbm, mi_hbm, mj_hbm,
          outw_hbm, outl_hbm,
          nnb_v, mj_v, nni_v, rm_v, wi_v, mi_v, aw_v, al_v):
        wid = lax.axis_index("s") * nc + lax.axis_index("c")
        base = wid * ROWS_PER_WORKER
        for pb in range(PB):
            pltpu.sync_copy(nnb_hbm.at[pb], nnb_v)
            pltpu.sync_copy(mj_hbm.at[pb], mj_v)
            pltpu.sync_copy(nni_hbm.at[pb, pl.ds(base, ROWS_PER_WORKER)], nni_v)
            pltpu.sync_copy(rmin_hbm.at[pb, pl.ds(base, ROWS_PER_WORKER)], rm_v)
            pltpu.sync_copy(w_hbm.at[2 * pb, pl.ds(base, ROWS_PER_WORKER)], wi_v)
            pltpu.sync_copy(mi_hbm.at[pb, pl.ds(base, ROWS_PER_WORKER)], mi_v)

            def chunk(c, carry):
                aw, al = carry
                sl = pl.ds(c * CHUNK, CHUNK)
                ids = nni_v[sl]
                nb = plsc.load_gather(nnb_v, [ids])
                mj = plsc.load_gather(mj_v, [ids])
                r = lax.iota(jnp.int32, CHUNK) + (base + c * CHUNK)
                valid = ((nb == r)
                         & (rm_v[sl] < THRESHOLD)
                         & (wi_v[sl] > 0.0))
                wv = jnp.where(valid, 1.0, 0.0).astype(jnp.float32)
                d = mi_v[sl] - mj
                return aw + wv, al + wv * d * d

            zero = jnp.zeros((CHUNK,), jnp.float32)
            aw, al = lax.fori_loop(0, ROWS_PER_WORKER // CHUNK, chunk,
                                   (zero, zero))
            aw_v[...] = aw
            al_v[...] = al
            pltpu.sync_copy(aw_v, outw_hbm.at[wid, pb])
            pltpu.sync_copy(al_v, outl_hbm.at[wid, pb])

    return k(nn_idx, nn_back, rmin, w_all, mask_i, mask_j)


def _finish(outw, outl, n_all):
    sw = outw.sum(axis=(0, 2))
    sl = outl.sum(axis=(0, 2))
    lpb = sl / jnp.maximum(sw, 1.0)
    inc = (n_all[0::2] >= 10) & (n_all[1::2] >= 10)
    lpb = jnp.where(inc, lpb, 0.0)
    lpair = lpb.reshape(NPAIR, B).sum(axis=1)
    cnt = inc.reshape(NPAIR, B).sum(axis=1)
    tot = jnp.where(cnt > 0, lpair / jnp.maximum(cnt, 1), 0.0).sum()
    return tot / NPAIR


def kernel(masks, pointmaps):
    tbls, idx_all, w_all, n_all = _setup(masks, pointmaps)
    pts_p, pts_t, mask_i, mask_j = _sc_gather(tbls, idx_all, w_all)
    rmin, ridx, nnb = _tc_nn(pts_p.reshape(PB, S, 4), pts_t)
    outw, outl = _sc_loss(ridx.reshape(PB, S), nnb.reshape(PB, S),
                          rmin.reshape(PB, S), w_all, mask_i, mask_j)
    return _finish(outw, outl, n_all)
